# baseline (speedup 1.0000x reference)
import functools

import jax
import jax.numpy as jnp
from jax import lax
from jax.experimental import pallas as pl
from jax.experimental.pallas import tpu as pltpu
from jax.experimental.pallas import tpu_sc as plsc

_NUM_CORES = 2
_NUM_SUBCORES = 16
_NUM_WORKERS = _NUM_CORES * _NUM_SUBCORES


def kernel(inputs, pos_embedding):
    seq_len = inputs.shape[1]
    emb_dim = pos_embedding.shape[1]
    rows_per_w = seq_len // _NUM_WORKERS

    mesh = plsc.VectorSubcoreMesh(core_axis_name="c", subcore_axis_name="s")

    @functools.partial(
        pl.kernel,
        out_type=jax.ShapeDtypeStruct((seq_len, emb_dim), pos_embedding.dtype),
        mesh=mesh,
    )
    def copy_rows(emb_hbm, out_hbm):
        wid = lax.axis_index("s") * _NUM_CORES + lax.axis_index("c")
        base = wid * rows_per_w
        pltpu.sync_copy(emb_hbm.at[pl.ds(base, rows_per_w)],
                        out_hbm.at[pl.ds(base, rows_per_w)])

    return copy_rows(pos_embedding)


# speedup vs baseline: 6.4436x; 6.4436x over previous
import functools

import jax
import jax.numpy as jnp
from jax import lax
from jax.experimental import pallas as pl
from jax.experimental.pallas import tpu as pltpu
from jax.experimental.pallas import tpu_sc as plsc

_NUM_CORES = 2
_NUM_SUBCORES = 16
_NUM_WORKERS = _NUM_CORES * _NUM_SUBCORES


def kernel(inputs, pos_embedding):
    seq_len = inputs.shape[1]
    emb_dim = pos_embedding.shape[1]
    rows_per_w = seq_len // _NUM_WORKERS
    nchunk = 8
    chunk = rows_per_w // nchunk

    mesh = plsc.VectorSubcoreMesh(core_axis_name="c", subcore_axis_name="s")

    @functools.partial(
        pl.kernel,
        out_type=jax.ShapeDtypeStruct((seq_len, emb_dim), pos_embedding.dtype),
        mesh=mesh,
        scratch_types=[
            pltpu.VMEM((rows_per_w, emb_dim), pos_embedding.dtype),
            [pltpu.SemaphoreType.DMA] * nchunk,
            [pltpu.SemaphoreType.DMA] * nchunk,
        ],
    )
    def copy_rows(emb_hbm, out_hbm, buf, rsems, wsems):
        wid = lax.axis_index("s") * _NUM_CORES + lax.axis_index("c")
        base = wid * rows_per_w
        reads = []
        for i in range(nchunk):
            c = pltpu.make_async_copy(
                emb_hbm.at[pl.ds(base + i * chunk, chunk)],
                buf.at[pl.ds(i * chunk, chunk)],
                rsems[i],
            )
            c.start()
            reads.append(c)
        writes = []
        for i in range(nchunk):
            reads[i].wait()
            w = pltpu.make_async_copy(
                buf.at[pl.ds(i * chunk, chunk)],
                out_hbm.at[pl.ds(base + i * chunk, chunk)],
                wsems[i],
            )
            w.start()
            writes.append(w)
        for w in writes:
            w.wait()

    return copy_rows(pos_embedding)


# speedup vs baseline: 6.5332x; 1.0139x over previous
import functools

import jax
import jax.numpy as jnp
from jax import lax
from jax.experimental import pallas as pl
from jax.experimental.pallas import tpu as pltpu
from jax.experimental.pallas import tpu_sc as plsc

_NUM_CORES = 2
_NUM_SUBCORES = 16
_NUM_WORKERS = _NUM_CORES * _NUM_SUBCORES


def kernel(inputs, pos_embedding):
    seq_len = inputs.shape[1]
    emb_dim = pos_embedding.shape[1]
    rows_per_w = seq_len // _NUM_CORES
    nchunk = 8
    chunk = rows_per_w // nchunk

    mesh = plsc.ScalarSubcoreMesh(axis_name="c")

    @functools.partial(
        pl.kernel,
        out_type=jax.ShapeDtypeStruct((seq_len, emb_dim), pos_embedding.dtype),
        mesh=mesh,
        scratch_types=[
            pltpu.VMEM_SHARED((rows_per_w, emb_dim), pos_embedding.dtype),
            [pltpu.SemaphoreType.DMA] * nchunk,
            [pltpu.SemaphoreType.DMA] * nchunk,
        ],
    )
    def copy_rows(emb_hbm, out_hbm, buf, rsems, wsems):
        wid = lax.axis_index("c")
        base = wid * rows_per_w
        reads = []
        for i in range(nchunk):
            c = pltpu.make_async_copy(
                emb_hbm.at[pl.ds(base + i * chunk, chunk)],
                buf.at[pl.ds(i * chunk, chunk)],
                rsems[i],
            )
            c.start()
            reads.append(c)
        writes = []
        for i in range(nchunk):
            reads[i].wait()
            w = pltpu.make_async_copy(
                buf.at[pl.ds(i * chunk, chunk)],
                out_hbm.at[pl.ds(base + i * chunk, chunk)],
                wsems[i],
            )
            w.start()
            writes.append(w)
        for w in writes:
            w.wait()

    return copy_rows(pos_embedding)


# speedup vs baseline: 6.5691x; 1.0055x over previous
import functools

import jax
import jax.numpy as jnp
from jax import lax
from jax.experimental import pallas as pl
from jax.experimental.pallas import tpu as pltpu
from jax.experimental.pallas import tpu_sc as plsc

_NUM_CORES = 2
_NUM_SUBCORES = 16
_NUM_WORKERS = _NUM_CORES * _NUM_SUBCORES


def kernel(inputs, pos_embedding):
    seq_len = inputs.shape[1]
    emb_dim = pos_embedding.shape[1]
    rows_per_w = seq_len // _NUM_WORKERS

    mesh = plsc.VectorSubcoreMesh(core_axis_name="c", subcore_axis_name="s")

    @functools.partial(
        pl.kernel,
        out_type=jax.ShapeDtypeStruct((seq_len, emb_dim), pos_embedding.dtype),
        mesh=mesh,
        scratch_types=[
            pltpu.VMEM((rows_per_w, emb_dim), pos_embedding.dtype),
        ],
    )
    def copy_rows(emb_hbm, out_hbm, buf):
        wid = lax.axis_index("s") * _NUM_CORES + lax.axis_index("c")
        base = wid * rows_per_w
        pltpu.sync_copy(emb_hbm.at[pl.ds(base, rows_per_w)], buf)
        pltpu.sync_copy(buf, out_hbm.at[pl.ds(base, rows_per_w)])

    return copy_rows(pos_embedding)
